# baseline (speedup 1.0000x reference)
import jax
import jax.numpy as jnp
from jax.experimental import pallas as pl
from jax.experimental.pallas import tpu as pltpu

_N_HEADS = 32
_HEAD_DIM = 128
_ROPE_DIM = 64
_TOPK = 2048
_BASE = 10000.0
_EPS = 1e-5


def _rope_ref(x):
    s = x.shape[-2]
    pos = jnp.arange(s, dtype=jnp.float32)
    freqs = _BASE ** (-jnp.arange(0, _ROPE_DIM, 2, dtype=jnp.float32) / _ROPE_DIM)
    theta = pos[:, None] * freqs[None, :]
    cos = jnp.cos(theta)
    sin = jnp.sin(theta)
    x1 = x[..., 0::2]
    x2 = x[..., 1::2]
    o1 = x1 * cos - x2 * sin
    o2 = x2 * cos + x1 * sin
    return jnp.stack([o1, o2], axis=-1).reshape(x.shape)


def _scores_body(q_ref, k_ref, w_ref, out_ref):
    k = k_ref[...]
    acc = None
    for h in range(_N_HEADS):
        qh = q_ref[:, h * _HEAD_DIM:(h + 1) * _HEAD_DIM]
        sc = jax.lax.dot_general(qh, k, (((1,), (1,)), ((), ())),
                                 preferred_element_type=jnp.float32)
        term = jnp.maximum(sc, 0.0) * w_ref[:, h:h + 1]
        acc = term if acc is None else acc + term
    out_ref[...] = acc


def _compute_scores(x, qr, W_qb, W_k, ln_g, ln_b, W_w):
    b, s, _ = x.shape
    q = (qr @ W_qb).reshape(b, s, _N_HEADS, _HEAD_DIM).transpose(0, 2, 1, 3)
    q_pe = _rope_ref(q[..., :_ROPE_DIM])
    q = jnp.concatenate([q_pe, q[..., _ROPE_DIM:]], axis=-1)
    q_flat = q[0].transpose(1, 0, 2).reshape(s, _N_HEADS * _HEAD_DIM)
    q_b = q_flat.astype(jnp.bfloat16)

    kx = x[0] @ W_k
    m = jnp.mean(kx, axis=-1, keepdims=True)
    v = jnp.var(kx, axis=-1, keepdims=True)
    k_ln = (kx - m) / jnp.sqrt(v + _EPS) * ln_g + ln_b
    k_pe = _rope_ref(k_ln[..., :_ROPE_DIM])
    k_rot = jnp.concatenate([k_pe, k_ln[..., _ROPE_DIM:]], axis=-1)
    k_b = k_rot.astype(jnp.bfloat16)
    w = (x[0] @ W_w) * (_N_HEADS ** -0.5 * _HEAD_DIM ** -0.5)

    QBLK = 256
    scores = pl.pallas_call(
        _scores_body,
        grid=(s // QBLK,),
        in_specs=[
            pl.BlockSpec((QBLK, _N_HEADS * _HEAD_DIM), lambda i: (i, 0)),
            pl.BlockSpec((s, _HEAD_DIM), lambda i: (0, 0)),
            pl.BlockSpec((QBLK, _N_HEADS), lambda i: (i, 0)),
        ],
        out_specs=pl.BlockSpec((QBLK, s), lambda i: (i, 0)),
        out_shape=jax.ShapeDtypeStruct((s, s), jnp.float32),
        compiler_params=pltpu.CompilerParams(
            dimension_semantics=("arbitrary",),
        ),
    )(q_b, k_b, w)
    return scores


def kernel(x, qr, mask, W_qb, W_k, ln_g, ln_b, W_w):
    scores = _compute_scores(x, qr, W_qb, W_k, ln_g, ln_b, W_w)
    _, idx = jax.lax.top_k(scores[None, None], _TOPK)
    return idx
